# Initial kernel scaffold; baseline (speedup 1.0000x reference)
import functools

import jax
import jax.numpy as jnp
from jax import lax
from jax.experimental import pallas as pl
from jax.experimental.pallas import tpu as pltpu
from jax.experimental.pallas import tpu_sc as plsc

_N = 10000
_E = 160000
_B = 125
_NB = 80
_RPT = 625
_BM = 1250


def _mm_body(x_ref, w_ref, o_ref):
    o_ref[...] = jnp.dot(x_ref[...], w_ref[...],
                         preferred_element_type=jnp.float32)


def _mm(x, w):
    m, k = x.shape
    n = w.shape[1]
    return pl.pallas_call(
        _mm_body,
        grid=(m // _BM,),
        in_specs=[pl.BlockSpec((_BM, k), lambda i: (i, 0)),
                  pl.BlockSpec((k, n), lambda i: (0, 0))],
        out_specs=pl.BlockSpec((_BM, n), lambda i: (i, 0)),
        out_shape=jax.ShapeDtypeStruct((m, n), jnp.float32),
    )(x, w)


def _mm_bias_relu_body(x_ref, b_ref, w_ref, o_ref):
    h = jnp.maximum(x_ref[...] + b_ref[...], 0.0)
    o_ref[...] = jnp.dot(h, w_ref[...], preferred_element_type=jnp.float32)


def _mm_bias_relu(x, b, w):
    m, k = x.shape
    n = w.shape[1]
    return pl.pallas_call(
        _mm_bias_relu_body,
        grid=(m // _BM,),
        in_specs=[pl.BlockSpec((_BM, k), lambda i: (i, 0)),
                  pl.BlockSpec((1, k), lambda i: (0, 0)),
                  pl.BlockSpec((k, n), lambda i: (0, 0))],
        out_specs=pl.BlockSpec((_BM, n), lambda i: (i, 0)),
        out_shape=jax.ShapeDtypeStruct((m, n), jnp.float32),
    )(x, b.reshape(1, k), w)


def _logsoftmax_bias_body(x_ref, b_ref, o_ref):
    h = x_ref[...] + b_ref[...]
    m = jnp.max(h, axis=1, keepdims=True)
    e = jnp.exp(h - m)
    s = jnp.sum(e, axis=1, keepdims=True)
    o_ref[...] = h - m - jnp.log(s)


def _logsoftmax_bias(x, b):
    m, n = x.shape
    return pl.pallas_call(
        _logsoftmax_bias_body,
        grid=(m // _BM,),
        in_specs=[pl.BlockSpec((_BM, n), lambda i: (i, 0)),
                  pl.BlockSpec((1, n), lambda i: (0, 0))],
        out_specs=pl.BlockSpec((_BM, n), lambda i: (i, 0)),
        out_shape=jax.ShapeDtypeStruct((m, n), jnp.float32),
    )(x, b.reshape(1, n))


_sc_mesh = plsc.VectorSubcoreMesh(core_axis_name="c", subcore_axis_name="s")


@functools.partial(
    pl.kernel,
    out_type=jax.ShapeDtypeStruct((2, _N, 128), jnp.float32),
    mesh=_sc_mesh,
    scratch_types=[
        pltpu.VMEM_SHARED((_N, 128), jnp.float32),
        pltpu.VMEM((_NB, _B), jnp.int32),
        pltpu.VMEM((_NB, _B), jnp.int32),
        pltpu.VMEM((_B, 128), jnp.float32),
        pltpu.SemaphoreType.DMA,
    ],
)
def _sc_spmm(sup2, col2, row3, zrows, out2, acc, colbuf, rowbuf, gbuf, sem):
    cid = lax.axis_index("c")
    sid = lax.axis_index("s")
    base = sid * _RPT
    pltpu.sync_copy(zrows, acc.at[pl.ds(base, _RPT)])
    pltpu.sync_copy(col2.at[cid, sid], colbuf)
    pltpu.sync_copy(row3.at[sid], rowbuf)
    plsc.subcore_barrier()

    def body(j, carry):
        pltpu.async_copy(sup2.at[colbuf.at[j]], gbuf, sem).wait()
        pltpu.sync_copy(gbuf, acc.at[rowbuf.at[j]], add=True)
        return carry

    lax.fori_loop(0, _NB, body, 0)
    plsc.subcore_barrier()
    pltpu.sync_copy(acc.at[pl.ds(base, _RPT)],
                    out2.at[cid, pl.ds(base, _RPT)])


def _spmm(sup, col2, row3, zrows):
    d = sup.shape[1]
    parts = []
    for c0 in range(0, d, 256):
        sup2 = jnp.concatenate([sup[:, c0:c0 + 128],
                                sup[:, c0 + 128:c0 + 256]], axis=0)
        out2 = _sc_spmm(sup2, col2, row3, zrows)
        parts.append(jnp.concatenate([out2[0], out2[1]], axis=1))
    return parts[0] if len(parts) == 1 else jnp.concatenate(parts, axis=1)


def kernel(edges, features, W1, b1, W2, b2, W3, b3):
    row = edges[0].astype(jnp.int32)
    col = edges[1].astype(jnp.int32)
    col2 = jnp.stack([col, col + _N]).reshape(2, 16, _NB, _B)
    row3 = row.reshape(16, _NB, _B)
    zrows = jnp.zeros((_RPT, 128), jnp.float32)

    sup = _mm(features, W1)
    agg = _spmm(sup, col2, row3, zrows)
    sup = _mm_bias_relu(agg, b1, W2)
    agg = _spmm(sup, col2, row3, zrows)
    sup = _mm_bias_relu(agg, b2, W3)
    agg = _spmm(sup, col2, row3, zrows)
    return _logsoftmax_bias(agg, b3)


# speedup vs baseline: 4.6479x; 4.6479x over previous
import functools

import jax
import jax.numpy as jnp
from jax import lax
from jax.experimental import pallas as pl
from jax.experimental.pallas import tpu as pltpu
from jax.experimental.pallas import tpu_sc as plsc

_N = 10000
_E = 160000
_B = 125
_NB = 80
_NP = 10112
_RPT = 632
_BM = 1000


def _mm_body(x_ref, w_ref, o_ref):
    o_ref[...] = jnp.dot(x_ref[...], w_ref[...],
                         preferred_element_type=jnp.float32)


def _mm(x, w):
    m, k = x.shape
    n = w.shape[1]
    return pl.pallas_call(
        _mm_body,
        grid=(m // _BM,),
        in_specs=[pl.BlockSpec((_BM, k), lambda i: (i, 0)),
                  pl.BlockSpec((k, n), lambda i: (0, 0))],
        out_specs=pl.BlockSpec((_BM, n), lambda i: (i, 0)),
        out_shape=jax.ShapeDtypeStruct((m, n), jnp.float32),
    )(x, w)


def _mm_bias_relu_body(x_ref, b_ref, w_ref, o_ref):
    h = jnp.maximum(x_ref[...] + b_ref[...], 0.0)
    o_ref[...] = jnp.dot(h, w_ref[...], preferred_element_type=jnp.float32)


def _mm_bias_relu(x, b, w):
    m, k = x.shape
    n = w.shape[1]
    return pl.pallas_call(
        _mm_bias_relu_body,
        grid=(m // _BM,),
        in_specs=[pl.BlockSpec((_BM, k), lambda i: (i, 0)),
                  pl.BlockSpec((1, k), lambda i: (0, 0)),
                  pl.BlockSpec((k, n), lambda i: (0, 0))],
        out_specs=pl.BlockSpec((_BM, n), lambda i: (i, 0)),
        out_shape=jax.ShapeDtypeStruct((m, n), jnp.float32),
    )(x, b.reshape(1, k), w)


def _logsoftmax_bias_body(x_ref, b_ref, o_ref):
    h = x_ref[...] + b_ref[...]
    m = jnp.max(h, axis=1, keepdims=True)
    e = jnp.exp(h - m)
    s = jnp.sum(e, axis=1, keepdims=True)
    o_ref[...] = h - m - jnp.log(s)


def _logsoftmax_bias(x, b):
    m, n = x.shape
    return pl.pallas_call(
        _logsoftmax_bias_body,
        grid=(m // _BM,),
        in_specs=[pl.BlockSpec((_BM, n), lambda i: (i, 0)),
                  pl.BlockSpec((1, n), lambda i: (0, 0))],
        out_specs=pl.BlockSpec((_BM, n), lambda i: (i, 0)),
        out_shape=jax.ShapeDtypeStruct((m, n), jnp.float32),
    )(x, b.reshape(1, n))


_sc_mesh = plsc.VectorSubcoreMesh(core_axis_name="c", subcore_axis_name="s",
                                  num_cores=2)


_sc_scratch = [
    pltpu.VMEM_SHARED((_NP, 128), jnp.float32),
    pltpu.VMEM((_NB, _B), jnp.int32),
    pltpu.VMEM((_NB, _B), jnp.int32),
    pltpu.VMEM((_B, 128), jnp.float32),
    pltpu.SemaphoreType.DMA,
]


def _sc_spmm_body(sup2, col2, row3, zrows, out2, acc, colbuf, rowbuf, gbuf, sem):
    cid = lax.axis_index("c")
    sid = lax.axis_index("s")
    base = sid * _RPT
    pltpu.sync_copy(zrows, acc.at[pl.ds(base, _RPT)])
    pltpu.sync_copy(col2.at[cid, sid], colbuf)
    pltpu.sync_copy(row3.at[sid], rowbuf)
    plsc.subcore_barrier()

    def body(j, carry):
        pltpu.async_copy(sup2.at[colbuf.at[j]], gbuf, sem).wait()
        pltpu.sync_copy(gbuf, acc.at[rowbuf.at[j]], add=True)
        return carry

    lax.fori_loop(0, _NB, body, 0)
    plsc.subcore_barrier()
    pltpu.sync_copy(acc.at[pl.ds(base, _RPT)],
                    out2.at[cid, pl.ds(base, _RPT)])


_sc_spmm = pl.kernel(
    _sc_spmm_body,
    out_type=jax.ShapeDtypeStruct((2, _NP, 128), jnp.float32),
    mesh=_sc_mesh,
    scratch_types=_sc_scratch,
)


def _spmm(sup, col2, row3, zrows):
    d = sup.shape[1]
    parts = []
    for c0 in range(0, d, 256):
        sup2 = jnp.concatenate([sup[:, c0:c0 + 128],
                                sup[:, c0 + 128:c0 + 256]], axis=0)
        out2 = _sc_spmm(sup2, col2, row3, zrows)
        parts.append(jnp.concatenate([out2[0, :_N], out2[1, :_N]], axis=1))
    return parts[0] if len(parts) == 1 else jnp.concatenate(parts, axis=1)


def kernel(edges, features, W1, b1, W2, b2, W3, b3):
    row = edges[0].astype(jnp.int32)
    col = edges[1].astype(jnp.int32)
    col2 = jnp.stack([col, col + _N]).reshape(2, 16, _NB, _B)
    row3 = row.reshape(16, _NB, _B)
    zrows = jnp.zeros((_RPT, 128), jnp.float32)

    sup = _mm(features, W1)
    agg = _spmm(sup, col2, row3, zrows)
    sup = _mm_bias_relu(agg, b1, W2)
    agg = _spmm(sup, col2, row3, zrows)
    sup = _mm_bias_relu(agg, b2, W3)
    agg = _spmm(sup, col2, row3, zrows)
    return _logsoftmax_bias(agg, b3)
